# Initial kernel scaffold; baseline (speedup 1.0000x reference)
import jax
import jax.numpy as jnp
from jax.experimental import pallas as pl


def kernel(node_feature, edge_index, emb_table, Wh, bh, Wfp, bfp, Wcl, bcl):
    raise NotImplementedError("write your pallas kernel here")


# speedup vs baseline: 4.4817x; 4.4817x over previous
import functools

import jax
import jax.numpy as jnp
from jax import lax
from jax.experimental import pallas as pl
from jax.experimental.pallas import tpu as pltpu
from jax.experimental.pallas import tpu_sc as plsc

N = 10000
NP = 10240
E = 320000
F = 128
NUM_FEAT = 128
NUM_CLASS = 10

NC = 2
NS = 16
CHUNK = 128
NCHUNKS = 79
EP = NC * NS * NCHUNKS * CHUNK
ROWS_PER_TILE = NP // NS


@functools.cache
def _make_sc_segment_sum():
    mesh = plsc.VectorSubcoreMesh(
        core_axis_name="c", subcore_axis_name="s", num_cores=NC, num_subcores=NS
    )

    @functools.partial(
        pl.kernel,
        out_type=jax.ShapeDtypeStruct((NC, NP, F), jnp.float32),
        mesh=mesh,
        scratch_types=[
            pltpu.VMEM((NCHUNKS, CHUNK), jnp.int32),
            pltpu.VMEM((NCHUNKS, CHUNK), jnp.int32),
            pltpu.VMEM((CHUNK, F), jnp.float32),
            pltpu.VMEM_SHARED((NP, F), jnp.float32),
            pltpu.SemaphoreType.DMA,
        ],
    )
    def sc_segment_sum(emb_hbm, src_hbm, dst_hbm, out_hbm,
                       src_v, dst_v, buf, acc_sh, sem):
        cid = lax.axis_index("c")
        sid = lax.axis_index("s")
        row0 = sid * ROWS_PER_TILE
        pltpu.sync_copy(emb_hbm.at[pl.ds(row0, ROWS_PER_TILE)],
                        acc_sh.at[pl.ds(row0, ROWS_PER_TILE)])
        pltpu.sync_copy(src_hbm.at[cid, sid], src_v)
        pltpu.sync_copy(dst_hbm.at[cid, sid], dst_v)
        plsc.subcore_barrier()

        @pl.loop(0, NCHUNKS)
        def _edge_chunk(j):
            pltpu.async_copy(emb_hbm.at[src_v.at[j]], buf, sem).wait()
            pltpu.sync_copy(buf, acc_sh.at[dst_v.at[j]], add=True)

        plsc.subcore_barrier()
        pltpu.sync_copy(acc_sh.at[pl.ds(row0, ROWS_PER_TILE)],
                        out_hbm.at[cid, pl.ds(row0, ROWS_PER_TILE)])

    return sc_segment_sum


def _sc_segment_sum(emb, src, dst):
    return _make_sc_segment_sum()(emb, src, dst)


_EMB_BLK = 1024


def _emb_body(ids_ref, table_ref, out_ref):
    ids = ids_ref[...]
    oh = (ids == lax.broadcasted_iota(jnp.int32, (_EMB_BLK, NUM_FEAT), 1))
    out_ref[...] = jnp.dot(oh.astype(jnp.float32), table_ref[...],
                           preferred_element_type=jnp.float32)


def _embed(node_feature, emb_table):
    ids = jnp.full((NP, 1), NUM_FEAT, dtype=jnp.int32)
    ids = ids.at[:N, 0].set(node_feature.astype(jnp.int32))
    return pl.pallas_call(
        _emb_body,
        grid=(NP // _EMB_BLK,),
        in_specs=[
            pl.BlockSpec((_EMB_BLK, 1), lambda i: (i, 0)),
            pl.BlockSpec((NUM_FEAT, F), lambda i: (0, 0)),
        ],
        out_specs=pl.BlockSpec((_EMB_BLK, F), lambda i: (i, 0)),
        out_shape=jax.ShapeDtypeStruct((NP, F), jnp.float32),
    )(ids, emb_table)


_DENSE_BLK = 1024


def _dense_body(p0_ref, p1_ref, emb_ref, wh_ref, bh_ref, wfp_ref, bfp_ref,
                r_ref, f_ref):
    i = pl.program_id(0)
    row = i * _DENSE_BLK + lax.broadcasted_iota(jnp.int32, (_DENSE_BLK, 1), 0)
    valid = (row < N).astype(jnp.float32)
    v = p0_ref[...] + p1_ref[...] - emb_ref[...]
    h = jnp.dot(v, wh_ref[...], preferred_element_type=jnp.float32) + bh_ref[...]
    h = jnp.maximum(h, 0.0) * valid
    r_ref[...] = h
    s = jnp.dot(h, wfp_ref[...], preferred_element_type=jnp.float32) + bfp_ref[...]
    s = s - jnp.max(s, axis=-1, keepdims=True)
    e = jnp.exp(s)
    p = e / jnp.sum(e, axis=-1, keepdims=True)

    @pl.when(i == 0)
    def _():
        f_ref[...] = jnp.zeros_like(f_ref)

    f_ref[...] += jnp.sum(p * valid, axis=0, keepdims=True)


def _dense_round(p0, p1, emb, wh, bh, wfp, bfp):
    return pl.pallas_call(
        _dense_body,
        grid=(NP // _DENSE_BLK,),
        in_specs=[
            pl.BlockSpec((_DENSE_BLK, F), lambda i: (i, 0)),
            pl.BlockSpec((_DENSE_BLK, F), lambda i: (i, 0)),
            pl.BlockSpec((_DENSE_BLK, F), lambda i: (i, 0)),
            pl.BlockSpec((F, F), lambda i: (0, 0)),
            pl.BlockSpec((1, F), lambda i: (0, 0)),
            pl.BlockSpec((F, F), lambda i: (0, 0)),
            pl.BlockSpec((1, F), lambda i: (0, 0)),
        ],
        out_specs=[
            pl.BlockSpec((_DENSE_BLK, F), lambda i: (i, 0)),
            pl.BlockSpec((1, F), lambda i: (0, 0)),
        ],
        out_shape=[
            jax.ShapeDtypeStruct((NP, F), jnp.float32),
            jax.ShapeDtypeStruct((1, F), jnp.float32),
        ],
    )(p0, p1, emb, wh, bh.reshape(1, F), wfp, bfp.reshape(1, F))


def _cls_body(f_ref, wcl_ref, bcl_ref, out_ref):
    s = jnp.dot(f_ref[...], wcl_ref[...],
                preferred_element_type=jnp.float32) + bcl_ref[...]
    s = s - jnp.max(s, axis=-1, keepdims=True)
    out_ref[...] = s - jnp.log(jnp.sum(jnp.exp(s), axis=-1, keepdims=True))


def _classify(f, wcl, bcl):
    out = pl.pallas_call(
        _cls_body,
        in_specs=[
            pl.BlockSpec((1, F), lambda: (0, 0)),
            pl.BlockSpec((F, NUM_CLASS), lambda: (0, 0)),
            pl.BlockSpec((1, NUM_CLASS), lambda: (0, 0)),
        ],
        out_specs=pl.BlockSpec((1, NUM_CLASS), lambda: (0, 0)),
        out_shape=jax.ShapeDtypeStruct((1, NUM_CLASS), jnp.float32),
    )(f, wcl, bcl.reshape(1, NUM_CLASS))
    return out.reshape(NUM_CLASS)


def kernel(node_feature, edge_index, emb_table, Wh, bh, Wfp, bfp, Wcl, bcl):
    src = jnp.zeros((EP,), jnp.int32).at[:E].set(edge_index[0].astype(jnp.int32))
    dst = jnp.full((EP,), N, jnp.int32).at[:E].set(edge_index[1].astype(jnp.int32))
    src = src.reshape(NC, NS, NCHUNKS, CHUNK)
    dst = dst.reshape(NC, NS, NCHUNKS, CHUNK)
    emb = _embed(node_feature, emb_table)
    f = jnp.zeros((1, F), dtype=jnp.float32)
    for l in range(3):
        partials = _sc_segment_sum(emb, src, dst)
        r, f_part = _dense_round(partials[0], partials[1], emb,
                                 Wh[l], bh[l], Wfp[l], bfp[l])
        f = f + f_part
        emb = r
    return _classify(f, Wcl, bcl)
